# baseline (speedup 1.0000x reference)
import functools

import jax
import jax.numpy as jnp
from jax import lax
from jax.experimental import pallas as pl
from jax.experimental.pallas import tpu as pltpu
from jax.experimental.pallas import tpu_sc as plsc

EMBED_DIM = 128
NUM_CORES = 2
NUM_SUBCORES = 16
NW = NUM_CORES * NUM_SUBCORES

CHUNK = 256
NBUF = 3


def _embed_gather(table, ids):
    B = ids.shape[0]
    rows_per_w = B // NW
    nchunk = rows_per_w // CHUNK

    mesh = plsc.VectorSubcoreMesh(core_axis_name="c", subcore_axis_name="s")

    @functools.partial(
        pl.kernel,
        mesh=mesh,
        out_type=jax.ShapeDtypeStruct((B, EMBED_DIM), jnp.float32),
        scratch_types=[
            pltpu.VMEM((NBUF * CHUNK,), jnp.int32),
            pltpu.VMEM((NBUF, CHUNK, EMBED_DIM), jnp.float32),
        ]
        + [pltpu.SemaphoreType.DMA] * (3 * NBUF),
    )
    def k(table_hbm, ids_hbm, out_hbm, idx_v, rows_v, *sems):
        gsem = list(sems[0:NBUF])
        ssem = list(sems[NBUF : 2 * NBUF])
        isem = list(sems[2 * NBUF : 3 * NBUF])
        wid = lax.axis_index("s") * NUM_CORES + lax.axis_index("c")
        base = wid * rows_per_w

        def idx_load(c, slot):
            pltpu.async_copy(
                ids_hbm.at[pl.ds(base + c * CHUNK, CHUNK)],
                idx_v.at[pl.ds(slot * CHUNK, CHUNK)], isem[slot],
            )

        def idx_wait(c, slot):
            pltpu.make_async_copy(
                ids_hbm.at[pl.ds(base + c * CHUNK, CHUNK)],
                idx_v.at[pl.ds(slot * CHUNK, CHUNK)], isem[slot],
            ).wait()

        def gather_start(slot):
            pltpu.async_copy(
                table_hbm.at[idx_v.at[pl.ds(slot * CHUNK, CHUNK)]], rows_v.at[slot], gsem[slot]
            )

        def gather_wait(slot):
            pltpu.make_async_copy(
                table_hbm.at[idx_v.at[pl.ds(slot * CHUNK, CHUNK)]], rows_v.at[slot], gsem[slot]
            ).wait()

        def store_start(c, slot):
            pltpu.async_copy(
                rows_v.at[slot],
                out_hbm.at[pl.ds(base + c * CHUNK, CHUNK)], ssem[slot],
            )

        def store_wait(c, slot):
            pltpu.make_async_copy(
                rows_v.at[slot],
                out_hbm.at[pl.ds(base + c * CHUNK, CHUNK)], ssem[slot],
            ).wait()

        idx_load(0, 0)
        idx_load(1, 1)
        for c in range(NBUF):
            idx_wait(c, c)
            gather_start(c)
            if c >= 1:
                gather_wait(c - 1)
                store_start(c - 1, c - 1)
            idx_load(c + 2, (c + 2) % NBUF)

        def body(g, carry):
            for b in range(NBUF):
                c = NBUF * g + b
                store_wait(c - NBUF, b)
                idx_wait(c, b)
                gather_start(b)
                prev = (b - 1) % NBUF
                gather_wait(prev)
                store_start(c - 1, prev)

                @pl.when(c + 2 < nchunk)
                def _(c=c, b=b):
                    idx_load(c + 2, (b + 2) % NBUF)

            return carry

        lax.fori_loop(1, (nchunk - 1) // NBUF, body, 0)

        last = nchunk - 1
        lb = last % NBUF
        store_wait(last - NBUF, lb)
        idx_wait(last, lb)
        gather_start(lb)
        gather_wait((lb - 1) % NBUF)
        store_start(last - 1, (lb - 1) % NBUF)
        gather_wait(lb)
        store_start(last, lb)
        for c in range(nchunk - NBUF, nchunk):
            store_wait(c, c % NBUF)

    return k(table, ids)


def kernel(input_ids, table):
    batch, seq = input_ids.shape
    ids = input_ids.reshape(-1)
    out = _embed_gather(table, ids)
    return out.reshape(batch, seq, EMBED_DIM)


# speedup vs baseline: 1.0094x; 1.0094x over previous
import functools

import jax
import jax.numpy as jnp
from jax import lax
from jax.experimental import pallas as pl
from jax.experimental.pallas import tpu as pltpu
from jax.experimental.pallas import tpu_sc as plsc

EMBED_DIM = 128
NUM_CORES = 2
NUM_SUBCORES = 16
NW = NUM_CORES * NUM_SUBCORES

CHUNK = 128
NBUF = 5


def _embed_gather(table, ids):
    B = ids.shape[0]
    rows_per_w = B // NW
    nchunk = rows_per_w // CHUNK

    mesh = plsc.VectorSubcoreMesh(core_axis_name="c", subcore_axis_name="s")

    @functools.partial(
        pl.kernel,
        mesh=mesh,
        out_type=jax.ShapeDtypeStruct((B, EMBED_DIM), jnp.float32),
        scratch_types=[
            pltpu.VMEM((NBUF, CHUNK), jnp.int32),
            pltpu.VMEM((NBUF, CHUNK, EMBED_DIM), jnp.float32),
        ]
        + [pltpu.SemaphoreType.DMA] * (3 * NBUF),
    )
    def k(table_hbm, ids_hbm, out_hbm, idx_v, rows_v, *sems):
        gsem = list(sems[0:NBUF])
        ssem = list(sems[NBUF : 2 * NBUF])
        isem = list(sems[2 * NBUF : 3 * NBUF])
        wid = lax.axis_index("s") * NUM_CORES + lax.axis_index("c")
        base = wid * rows_per_w

        def idx_load(c, slot):
            pltpu.async_copy(
                ids_hbm.at[pl.ds(base + c * CHUNK, CHUNK)],
                idx_v.at[slot], isem[slot],
            )

        def idx_wait(c, slot):
            pltpu.make_async_copy(
                ids_hbm.at[pl.ds(base + c * CHUNK, CHUNK)],
                idx_v.at[slot], isem[slot],
            ).wait()

        def gather_start(slot):
            pltpu.async_copy(
                table_hbm.at[idx_v.at[slot]], rows_v.at[slot], gsem[slot]
            )

        def gather_wait(slot):
            pltpu.make_async_copy(
                table_hbm.at[idx_v.at[slot]], rows_v.at[slot], gsem[slot]
            ).wait()

        def store_start(c, slot):
            pltpu.async_copy(
                rows_v.at[slot],
                out_hbm.at[pl.ds(base + c * CHUNK, CHUNK)], ssem[slot],
            )

        def store_wait(c, slot):
            pltpu.make_async_copy(
                rows_v.at[slot],
                out_hbm.at[pl.ds(base + c * CHUNK, CHUNK)], ssem[slot],
            ).wait()

        idx_load(0, 0)
        idx_load(1, 1)
        for c in range(NBUF):
            idx_wait(c, c)
            gather_start(c)
            if c >= 3:
                gather_wait(c - 3)
                store_start(c - 3, c - 3)
            idx_load(c + 2, (c + 2) % NBUF)

        def body(g, carry):
            for b in range(NBUF):
                c = NBUF * g + b
                store_wait(c - NBUF, b)
                idx_wait(c, b)
                gather_start(b)
                prev3 = (b - 3) % NBUF
                gather_wait(prev3)
                store_start(c - 3, prev3)

                @pl.when(c + 2 < nchunk)
                def _(c=c, b=b):
                    idx_load(c + 2, (b + 2) % NBUF)

            return carry

        lax.fori_loop(1, nchunk // NBUF, body, 0)

        for c in range(nchunk - 3, nchunk):
            gather_wait(c % NBUF)
            store_start(c, c % NBUF)
        for c in range(nchunk - NBUF, nchunk):
            store_wait(c, c % NBUF)

    return k(table, ids)


def kernel(input_ids, table):
    batch, seq = input_ids.shape
    ids = input_ids.reshape(-1)
    out = _embed_gather(table, ids)
    return out.reshape(batch, seq, EMBED_DIM)
